# baseline (speedup 1.0000x reference)
import functools

import jax
import jax.numpy as jnp
import numpy as np
from jax import lax
from jax.experimental import pallas as pl
from jax.experimental.pallas import tpu as pltpu
from jax.experimental.pallas import tpu_sc as plsc

K = 1.0
R0 = 1.0

NC = 2
NS = 16
NW = NC * NS
L = 16

_MAGIC = np.int32(0x5F3759DF)
_TINY = np.float32(1e-35)
_HIMASK = np.int32(-65536)


def _pick_block(per_tile: int, cap: int = 2048) -> int:
    for b in range(cap, 15, -16):
        if per_tile % b == 0 and (per_tile // b) % 4 == 0 and per_tile // b >= 8:
            return b
    raise ValueError(f"no pipelined block size divides {per_tile}")


@functools.lru_cache(maxsize=None)
def _build_sc_kernel(n_pad: int, n_edges: int):
    per_tile = n_edges // NW
    B = _pick_block(per_tile)
    n_blocks = per_tile // B
    chunk = n_pad // NS

    mesh = plsc.VectorSubcoreMesh(core_axis_name="c", subcore_axis_name="s")

    def body(px, pyz, eflat, fout, eout, *refs):
        (sx, syz, sfx, sfy, sfz) = refs[0:5]
        idxs = refs[5:9]
        gsets = (refs[9:11], refs[11:13])
        fsets = (refs[13:16], refs[16:19])
        stage, acc_vm = refs[19], refs[20]
        semi, semg, sems = refs[21], refs[22], refs[23]

        c = lax.axis_index("c")
        s = lax.axis_index("s")
        wid = c * NS + s

        zeros = jnp.zeros((L,), jnp.float32)

        def zbody(t, carry):
            stage[pl.ds(t * L, L)] = zeros
            return carry

        lax.fori_loop(0, chunk // L, zbody, 0)

        off = s * chunk
        nsl = pl.ds(off, chunk)
        pltpu.sync_copy(stage, sfx.at[nsl])
        pltpu.sync_copy(stage, sfy.at[nsl])
        pltpu.sync_copy(stage, sfz.at[nsl])
        pltpu.sync_copy(px.at[nsl], stage)
        pltpu.sync_copy(stage, sx.at[nsl])
        half = chunk // 2
        for q in range(2):
            hsl = pl.ds(off + q * half, half)
            pltpu.sync_copy(pyz.at[hsl], idxs[0].at[pl.ds(0, half)])
            pltpu.sync_copy(idxs[0].at[pl.ds(0, half)], syz.at[hsl])
        plsc.subcore_barrier()

        ebase = wid * per_tile

        def issue_idx(b, p4):
            base = ebase + b * B
            pltpu.async_copy(eflat.at[pl.ds(base, B)],
                             idxs[p4].at[pl.ds(0, B)], semi)
            pltpu.async_copy(eflat.at[pl.ds(n_edges + base, B)],
                             idxs[p4].at[pl.ds(B, B)], semi)

        def drain_idx(p4):
            pltpu.make_async_copy(eflat.at[pl.ds(0, B)],
                                  idxs[p4].at[pl.ds(0, B)], semi).wait()
            pltpu.make_async_copy(eflat.at[pl.ds(0, B)],
                                  idxs[p4].at[pl.ds(B, B)], semi).wait()

        def issue_gathers(p4, p2):
            gx, gyz = gsets[p2]
            pltpu.async_copy(sx.at[idxs[p4]], gx, semg)
            pltpu.async_copy(syz.at[idxs[p4]], gyz, semg)

        def drain_gathers(p2):
            gx, gyz = gsets[p2]
            pltpu.make_async_copy(px.at[pl.ds(0, 2 * B)], gx, semg).wait()
            pltpu.make_async_copy(eflat.at[pl.ds(0, 2 * B)], gyz,
                                  semg).wait()

        def issue_scatters(p4, p2):
            fx, fy, fz = fsets[p2]
            pltpu.async_copy(fx, sfx.at[idxs[p4]], sems, add=True)
            pltpu.async_copy(fy, sfy.at[idxs[p4]], sems, add=True)
            pltpu.async_copy(fz, sfz.at[idxs[p4]], sems, add=True)

        def drain_scatters(p2):
            for buf in fsets[p2]:
                pltpu.make_async_copy(px.at[pl.ds(0, 2 * B)], buf,
                                      sems).wait()

        def compute(p2, acc):
            gx, gyz = gsets[p2]
            fbx, fby, fbz = fsets[p2]

            def step(t, acc):
                sl = pl.ds(t * L, L)
                sj = pl.ds(B + t * L, L)
                wi = gyz[sl]
                wj = gyz[sj]
                yi = lax.bitcast_convert_type(wi << 16, jnp.float32)
                yj = lax.bitcast_convert_type(wj << 16, jnp.float32)
                zi = lax.bitcast_convert_type(wi & _HIMASK, jnp.float32)
                zj = lax.bitcast_convert_type(wj & _HIMASK, jnp.float32)
                dx = gx[sl] - gx[sj]
                dy = yi - yj
                dz = zi - zj
                x2 = dx * dx + dy * dy + dz * dz
                x2c = jnp.maximum(x2, _TINY)
                h = jnp.float32(0.5) * x2c
                y = lax.bitcast_convert_type(
                    _MAGIC - (lax.bitcast_convert_type(x2c, jnp.int32) >> 1),
                    jnp.float32)
                y = y * (jnp.float32(1.5) - h * y * y)
                y = y * (jnp.float32(1.5) - h * y * y)
                d = x2 * y
                delta = d - jnp.float32(R0)
                acc = acc + delta * delta
                sf = jnp.float32(K) * delta * y
                fx = sf * dx
                fy = sf * dy
                fz = sf * dz
                fbx[sj] = fx
                fby[sj] = fy
                fbz[sj] = fz
                fbx[sl] = -fx
                fby[sl] = -fy
                fbz[sl] = -fz
                return acc

            return lax.fori_loop(0, B // L, step, acc)

        def do_block(b, p4, p2, first, has_next, has_next2, acc):
            drain_gathers(p2)
            if has_next:
                drain_idx((p4 + 1) % 4)
                issue_gathers((p4 + 1) % 4, (p2 + 1) % 2)
            if has_next2:
                issue_idx(b + 2, (p4 + 2) % 4)
            acc = compute(p2, acc)
            if not first:
                drain_scatters((p2 + 1) % 2)
            issue_scatters(p4, p2)
            return acc

        pltpu.sync_copy(eflat.at[pl.ds(ebase, B)], idxs[0].at[pl.ds(0, B)])
        pltpu.sync_copy(eflat.at[pl.ds(n_edges + ebase, B)],
                        idxs[0].at[pl.ds(B, B)])
        issue_gathers(0, 0)
        issue_idx(1, 1)

        acc = jnp.zeros((L,), jnp.float32)
        acc = do_block(0, 0, 0, True, True, True, acc)
        acc = do_block(1, 1, 1, False, True, True, acc)

        def loop_body(m, acc):
            b0 = 2 + m * 4
            for k in range(4):
                acc = do_block(b0 + k, (2 + k) % 4, k % 2,
                               False, True, True, acc)
            return acc

        acc = lax.fori_loop(0, (n_blocks - 4) // 4, loop_body, acc)
        acc = do_block(n_blocks - 2, (n_blocks - 2) % 4, (n_blocks - 2) % 2,
                       False, True, False, acc)
        acc = do_block(n_blocks - 1, (n_blocks - 1) % 4, (n_blocks - 1) % 2,
                       False, False, False, acc)
        drain_scatters((n_blocks - 1) % 2)

        acc_vm[...] = acc
        pltpu.sync_copy(acc_vm, eout.at[pl.ds(wid * L, L)])

        plsc.subcore_barrier()
        fbase = c * 3 * n_pad + off
        pltpu.sync_copy(sfx.at[nsl], stage)
        pltpu.sync_copy(stage, fout.at[pl.ds(fbase, chunk)])
        pltpu.sync_copy(sfy.at[nsl], stage)
        pltpu.sync_copy(stage, fout.at[pl.ds(fbase + n_pad, chunk)])
        pltpu.sync_copy(sfz.at[nsl], stage)
        pltpu.sync_copy(stage, fout.at[pl.ds(fbase + 2 * n_pad, chunk)])

    f32 = jnp.float32
    i32 = jnp.int32
    scratch = (
        [pltpu.VMEM_SHARED((n_pad,), f32)]
        + [pltpu.VMEM_SHARED((n_pad,), i32)]
        + [pltpu.VMEM_SHARED((n_pad,), f32)] * 3
        + [pltpu.VMEM((2 * B,), i32)] * 4
        + [pltpu.VMEM((2 * B,), f32), pltpu.VMEM((2 * B,), i32)] * 2
        + [pltpu.VMEM((2 * B,), f32)] * 6
        + [pltpu.VMEM((chunk,), f32)]
        + [pltpu.VMEM((L,), f32)]
        + [pltpu.SemaphoreType.DMA] * 3
    )
    return pl.kernel(
        body,
        out_type=[
            jax.ShapeDtypeStruct((NC * 3 * n_pad,), f32),
            jax.ShapeDtypeStruct((NW * L,), f32),
        ],
        mesh=mesh,
        scratch_types=scratch,
    )


def kernel(pos, edge_index):
    n, _ = pos.shape
    e = edge_index.shape[1]
    align = NS * L
    n_pad = ((n + align - 1) // align) * align
    posp = jnp.pad(pos, ((0, n_pad - n), (0, 0)))
    px = posp[:, 0]
    yu = lax.bitcast_convert_type(posp[:, 1], jnp.uint32)
    zu = lax.bitcast_convert_type(posp[:, 2], jnp.uint32)
    rnd = jnp.uint32(0x8000)
    yz = ((zu + rnd) & jnp.uint32(0xFFFF0000)) | ((yu + rnd) >> 16)
    pyz = lax.bitcast_convert_type(yz, jnp.int32)
    eflat = edge_index.reshape(2 * e)
    sck = _build_sc_kernel(n_pad, e)
    fout, eout = sck(px, pyz, eflat)
    fout = fout.reshape(NC, 3, n_pad)
    forces = (fout[0] + fout[1])[:, :n].T
    energy = (0.5 * K) * jnp.sum(eout).reshape(1)
    return energy, forces


# speedup vs baseline: 1.3789x; 1.3789x over previous
import functools

import jax
import jax.numpy as jnp
import numpy as np
from jax import lax
from jax.experimental import pallas as pl
from jax.experimental.pallas import tpu as pltpu
from jax.experimental.pallas import tpu_sc as plsc

K = 1.0
R0 = 1.0

NC = 2
NS = 16
NW = NC * NS
L = 16

_MAGIC = np.int32(0x5F3759DF)
_TINY = np.float32(1e-35)


def _pick_block(per_tile: int, cap: int = 2048) -> int:
    for b in range(cap, 15, -16):
        if per_tile % b == 0 and (per_tile // b) % 4 == 0 and per_tile // b >= 8:
            return b
    raise ValueError(f"no pipelined block size divides {per_tile}")


@functools.lru_cache(maxsize=None)
def _build_sc_kernel(n_pad: int, n_edges: int):
    per_tile = n_edges // NW
    B = _pick_block(per_tile)
    n_blocks = per_tile // B
    chunk = n_pad // NS

    mesh = plsc.VectorSubcoreMesh(core_axis_name="c", subcore_axis_name="s")

    def body(px, py, pz, eflat, fout, eout, *refs):
        (sx, sy, sz, sfx, sfy, sfz) = refs[0:6]
        idxs = refs[6:10]
        gsets = (refs[10:13], refs[13:16])
        fsets = (refs[16:19], refs[19:22])
        stage, acc_vm = refs[22], refs[23]
        semi, semg, sems = refs[24], refs[25], refs[26]

        c = lax.axis_index("c")
        s = lax.axis_index("s")
        wid = c * NS + s

        zeros = jnp.zeros((L,), jnp.float32)

        def zbody(t, carry):
            stage[pl.ds(t * L, L)] = zeros
            return carry

        lax.fori_loop(0, chunk // L, zbody, 0)

        off = s * chunk
        nsl = pl.ds(off, chunk)
        pltpu.sync_copy(stage, sfx.at[nsl])
        pltpu.sync_copy(stage, sfy.at[nsl])
        pltpu.sync_copy(stage, sfz.at[nsl])
        pltpu.sync_copy(px.at[nsl], stage)
        pltpu.sync_copy(stage, sx.at[nsl])
        pltpu.sync_copy(py.at[nsl], stage)
        pltpu.sync_copy(stage, sy.at[nsl])
        pltpu.sync_copy(pz.at[nsl], stage)
        pltpu.sync_copy(stage, sz.at[nsl])
        plsc.subcore_barrier()

        ebase = wid * per_tile

        def issue_idx(b, p4):
            base = ebase + b * B
            pltpu.async_copy(eflat.at[pl.ds(base, B)],
                             idxs[p4].at[pl.ds(0, B)], semi)
            pltpu.async_copy(eflat.at[pl.ds(n_edges + base, B)],
                             idxs[p4].at[pl.ds(B, B)], semi)

        def drain_idx(p4):
            pltpu.make_async_copy(eflat.at[pl.ds(0, B)],
                                  idxs[p4].at[pl.ds(0, B)], semi).wait()
            pltpu.make_async_copy(eflat.at[pl.ds(0, B)],
                                  idxs[p4].at[pl.ds(B, B)], semi).wait()

        def issue_gathers(p4, p2):
            gx, gy, gz = gsets[p2]
            pltpu.async_copy(sx.at[idxs[p4]], gx, semg)
            pltpu.async_copy(sy.at[idxs[p4]], gy, semg)
            pltpu.async_copy(sz.at[idxs[p4]], gz, semg)

        def drain_gathers(p2):
            for buf in gsets[p2]:
                pltpu.make_async_copy(px.at[pl.ds(0, 2 * B)], buf,
                                      semg).wait()

        def issue_scatters(p4, p2):
            fx, fy, fz = fsets[p2]
            pltpu.async_copy(fx, sfx.at[idxs[p4]], sems, add=True)
            pltpu.async_copy(fy, sfy.at[idxs[p4]], sems, add=True)
            pltpu.async_copy(fz, sfz.at[idxs[p4]], sems, add=True)

        def drain_scatters(p2):
            for buf in fsets[p2]:
                pltpu.make_async_copy(px.at[pl.ds(0, 2 * B)], buf,
                                      sems).wait()

        def compute(p2, acc):
            gx, gy, gz = gsets[p2]
            fbx, fby, fbz = fsets[p2]
            UNROLL = 4

            def step(t, acc):
                for k in range(UNROLL):
                    sl = pl.ds(t * (L * UNROLL) + k * L, L)
                    sj = pl.ds(B + t * (L * UNROLL) + k * L, L)
                    dx = gx[sl] - gx[sj]
                    dy = gy[sl] - gy[sj]
                    dz = gz[sl] - gz[sj]
                    x2 = dx * dx + dy * dy + dz * dz
                    x2c = jnp.maximum(x2, _TINY)
                    h = jnp.float32(0.5) * x2c
                    y = lax.bitcast_convert_type(
                        _MAGIC
                        - (lax.bitcast_convert_type(x2c, jnp.int32) >> 1),
                        jnp.float32)
                    y = y * (jnp.float32(1.5) - h * y * y)
                    y = y * (jnp.float32(1.5) - h * y * y)
                    d = x2 * y
                    delta = d - jnp.float32(R0)
                    acc = acc + delta * delta
                    sf = jnp.float32(K) * delta * y
                    fx = sf * dx
                    fy = sf * dy
                    fz = sf * dz
                    fbx[sj] = fx
                    fby[sj] = fy
                    fbz[sj] = fz
                    fbx[sl] = -fx
                    fby[sl] = -fy
                    fbz[sl] = -fz
                return acc

            return lax.fori_loop(0, B // (L * UNROLL), step, acc)

        def do_block(b, p4, p2, first, has_next, has_next2, acc):
            drain_gathers(p2)
            if has_next:
                drain_idx((p4 + 1) % 4)
                issue_gathers((p4 + 1) % 4, (p2 + 1) % 2)
            if has_next2:
                issue_idx(b + 2, (p4 + 2) % 4)
            acc = compute(p2, acc)
            if not first:
                drain_scatters((p2 + 1) % 2)
            issue_scatters(p4, p2)
            return acc

        pltpu.sync_copy(eflat.at[pl.ds(ebase, B)], idxs[0].at[pl.ds(0, B)])
        pltpu.sync_copy(eflat.at[pl.ds(n_edges + ebase, B)],
                        idxs[0].at[pl.ds(B, B)])
        issue_gathers(0, 0)
        issue_idx(1, 1)

        acc = jnp.zeros((L,), jnp.float32)
        acc = do_block(0, 0, 0, True, True, True, acc)
        acc = do_block(1, 1, 1, False, True, True, acc)

        def loop_body(m, acc):
            b0 = 2 + m * 4
            for k in range(4):
                acc = do_block(b0 + k, (2 + k) % 4, k % 2,
                               False, True, True, acc)
            return acc

        acc = lax.fori_loop(0, (n_blocks - 4) // 4, loop_body, acc)
        acc = do_block(n_blocks - 2, (n_blocks - 2) % 4, (n_blocks - 2) % 2,
                       False, True, False, acc)
        acc = do_block(n_blocks - 1, (n_blocks - 1) % 4, (n_blocks - 1) % 2,
                       False, False, False, acc)
        drain_scatters((n_blocks - 1) % 2)

        acc_vm[...] = acc
        pltpu.sync_copy(acc_vm, eout.at[pl.ds(wid * L, L)])

        plsc.subcore_barrier()
        fbase = c * 3 * n_pad + off
        pltpu.sync_copy(sfx.at[nsl], stage)
        pltpu.sync_copy(stage, fout.at[pl.ds(fbase, chunk)])
        pltpu.sync_copy(sfy.at[nsl], stage)
        pltpu.sync_copy(stage, fout.at[pl.ds(fbase + n_pad, chunk)])
        pltpu.sync_copy(sfz.at[nsl], stage)
        pltpu.sync_copy(stage, fout.at[pl.ds(fbase + 2 * n_pad, chunk)])

    f32 = jnp.float32
    i32 = jnp.int32
    scratch = (
        [pltpu.VMEM_SHARED((n_pad,), f32)] * 6
        + [pltpu.VMEM((2 * B,), i32)] * 4
        + [pltpu.VMEM((2 * B,), f32)] * 6
        + [pltpu.VMEM((2 * B,), f32)] * 6
        + [pltpu.VMEM((chunk,), f32)]
        + [pltpu.VMEM((L,), f32)]
        + [pltpu.SemaphoreType.DMA] * 3
    )
    return pl.kernel(
        body,
        out_type=[
            jax.ShapeDtypeStruct((NC * 3 * n_pad,), f32),
            jax.ShapeDtypeStruct((NW * L,), f32),
        ],
        mesh=mesh,
        scratch_types=scratch,
    )


def kernel(pos, edge_index):
    n, _ = pos.shape
    e = edge_index.shape[1]
    align = NS * L
    n_pad = ((n + align - 1) // align) * align
    posp = jnp.pad(pos, ((0, n_pad - n), (0, 0)))
    px = posp[:, 0]
    py = posp[:, 1]
    pz = posp[:, 2]
    eflat = edge_index.reshape(2 * e)
    sck = _build_sc_kernel(n_pad, e)
    fout, eout = sck(px, py, pz, eflat)
    fout = fout.reshape(NC, 3, n_pad)
    forces = (fout[0] + fout[1])[:, :n].T
    energy = (0.5 * K) * jnp.sum(eout).reshape(1)
    return energy, forces


# speedup vs baseline: 1.3800x; 1.0009x over previous
import functools

import jax
import jax.numpy as jnp
import numpy as np
from jax import lax
from jax.experimental import pallas as pl
from jax.experimental.pallas import tpu as pltpu
from jax.experimental.pallas import tpu_sc as plsc

K = 1.0
R0 = 1.0

NC = 2
NS = 16
NW = NC * NS
L = 16

_MAGIC = np.int32(0x5F3759DF)
_TINY = np.float32(1e-35)


def _pick_block(per_tile: int, cap: int = 2048) -> int:
    for b in range(cap, 15, -16):
        if per_tile % b == 0 and (per_tile // b) % 4 == 0 and per_tile // b >= 8:
            return b
    raise ValueError(f"no pipelined block size divides {per_tile}")


@functools.lru_cache(maxsize=None)
def _build_sc_kernel(n_pad: int, n_edges: int):
    per_tile = n_edges // NW
    B = _pick_block(per_tile)
    n_blocks = per_tile // B
    chunk = n_pad // NS

    mesh = plsc.VectorSubcoreMesh(core_axis_name="c", subcore_axis_name="s")

    def body(px, py, pz, eflat, fout, eout, *refs):
        (sx, sy, sz, sfx, sfy, sfz) = refs[0:6]
        idxs = refs[6:10]
        gsets = (refs[10:13], refs[13:16])
        fsets = (refs[16:19], refs[19:22])
        stage, acc_vm = refs[22], refs[23]
        semi, semg, sems = refs[24], refs[25], refs[26]

        c = lax.axis_index("c")
        s = lax.axis_index("s")
        wid = c * NS + s

        zeros = jnp.zeros((L,), jnp.float32)

        def zbody(t, carry):
            stage[pl.ds(t * L, L)] = zeros
            return carry

        lax.fori_loop(0, chunk // L, zbody, 0)

        off = s * chunk
        nsl = pl.ds(off, chunk)
        pltpu.sync_copy(stage, sfx.at[nsl])
        pltpu.sync_copy(stage, sfy.at[nsl])
        pltpu.sync_copy(stage, sfz.at[nsl])
        pltpu.sync_copy(px.at[nsl], stage)
        pltpu.sync_copy(stage, sx.at[nsl])
        pltpu.sync_copy(py.at[nsl], stage)
        pltpu.sync_copy(stage, sy.at[nsl])
        pltpu.sync_copy(pz.at[nsl], stage)
        pltpu.sync_copy(stage, sz.at[nsl])
        plsc.subcore_barrier()

        ebase = wid * per_tile

        def issue_idx(b, p4):
            base = ebase + b * B
            pltpu.async_copy(eflat.at[pl.ds(base, B)],
                             idxs[p4].at[pl.ds(0, B)], semi)
            pltpu.async_copy(eflat.at[pl.ds(n_edges + base, B)],
                             idxs[p4].at[pl.ds(B, B)], semi)

        def drain_idx(p4):
            pltpu.make_async_copy(eflat.at[pl.ds(0, B)],
                                  idxs[p4].at[pl.ds(0, B)], semi).wait()
            pltpu.make_async_copy(eflat.at[pl.ds(0, B)],
                                  idxs[p4].at[pl.ds(B, B)], semi).wait()

        def issue_gathers(p4, p2):
            gx, gy, gz = gsets[p2]
            pltpu.async_copy(sx.at[idxs[p4]], gx, semg)
            pltpu.async_copy(sy.at[idxs[p4]], gy, semg)
            pltpu.async_copy(sz.at[idxs[p4]], gz, semg)

        def drain_gathers(p2):
            for buf in gsets[p2]:
                pltpu.make_async_copy(px.at[pl.ds(0, 2 * B)], buf,
                                      semg).wait()

        def issue_scatters(p4, p2):
            fx, fy, fz = fsets[p2]
            pltpu.async_copy(fx, sfx.at[idxs[p4]], sems, add=True)
            pltpu.async_copy(fy, sfy.at[idxs[p4]], sems, add=True)
            pltpu.async_copy(fz, sfz.at[idxs[p4]], sems, add=True)

        def drain_scatters(p2):
            for buf in fsets[p2]:
                pltpu.make_async_copy(px.at[pl.ds(0, 2 * B)], buf,
                                      sems).wait()

        def compute(p2, acc):
            gx, gy, gz = gsets[p2]
            fbx, fby, fbz = fsets[p2]

            def step(t, acc):
                sl = pl.ds(t * L, L)
                sj = pl.ds(B + t * L, L)
                dx = gx[sl] - gx[sj]
                dy = gy[sl] - gy[sj]
                dz = gz[sl] - gz[sj]
                x2 = dx * dx + dy * dy + dz * dz
                x2c = jnp.maximum(x2, _TINY)
                h = jnp.float32(0.5) * x2c
                y = lax.bitcast_convert_type(
                    _MAGIC - (lax.bitcast_convert_type(x2c, jnp.int32) >> 1),
                    jnp.float32)
                y = y * (jnp.float32(1.5) - h * y * y)
                y = y * (jnp.float32(1.5) - h * y * y)
                d = x2 * y
                delta = d - jnp.float32(R0)
                acc = acc + delta * delta
                sf = jnp.float32(K) * delta * y
                fx = sf * dx
                fy = sf * dy
                fz = sf * dz
                fbx[sj] = fx
                fby[sj] = fy
                fbz[sj] = fz
                fbx[sl] = -fx
                fby[sl] = -fy
                fbz[sl] = -fz
                return acc

            return lax.fori_loop(0, B // L, step, acc)

        def do_block(b, p4, p2, first, has_next, has_next2, acc):
            drain_gathers(p2)
            if has_next:
                drain_idx((p4 + 1) % 4)
                issue_gathers((p4 + 1) % 4, (p2 + 1) % 2)
            if has_next2:
                issue_idx(b + 2, (p4 + 2) % 4)
            acc = compute(p2, acc)
            if not first:
                drain_scatters((p2 + 1) % 2)
            issue_scatters(p4, p2)
            return acc

        pltpu.sync_copy(eflat.at[pl.ds(ebase, B)], idxs[0].at[pl.ds(0, B)])
        pltpu.sync_copy(eflat.at[pl.ds(n_edges + ebase, B)],
                        idxs[0].at[pl.ds(B, B)])
        issue_gathers(0, 0)
        issue_idx(1, 1)

        acc = jnp.zeros((L,), jnp.float32)
        acc = do_block(0, 0, 0, True, True, True, acc)
        acc = do_block(1, 1, 1, False, True, True, acc)

        def loop_body(m, acc):
            b0 = 2 + m * 4
            for k in range(4):
                acc = do_block(b0 + k, (2 + k) % 4, k % 2,
                               False, True, True, acc)
            return acc

        acc = lax.fori_loop(0, (n_blocks - 4) // 4, loop_body, acc)
        acc = do_block(n_blocks - 2, (n_blocks - 2) % 4, (n_blocks - 2) % 2,
                       False, True, False, acc)
        acc = do_block(n_blocks - 1, (n_blocks - 1) % 4, (n_blocks - 1) % 2,
                       False, False, False, acc)
        drain_scatters((n_blocks - 1) % 2)

        acc_vm[...] = acc
        pltpu.sync_copy(acc_vm, eout.at[pl.ds(wid * L, L)])

        plsc.subcore_barrier()
        fbase = c * 3 * n_pad + off
        pltpu.sync_copy(sfx.at[nsl], stage)
        pltpu.sync_copy(stage, fout.at[pl.ds(fbase, chunk)])
        pltpu.sync_copy(sfy.at[nsl], stage)
        pltpu.sync_copy(stage, fout.at[pl.ds(fbase + n_pad, chunk)])
        pltpu.sync_copy(sfz.at[nsl], stage)
        pltpu.sync_copy(stage, fout.at[pl.ds(fbase + 2 * n_pad, chunk)])

    f32 = jnp.float32
    i32 = jnp.int32
    scratch = (
        [pltpu.VMEM_SHARED((n_pad,), f32)] * 6
        + [pltpu.VMEM((2 * B,), i32)] * 4
        + [pltpu.VMEM((2 * B,), f32)] * 6
        + [pltpu.VMEM((2 * B,), f32)] * 6
        + [pltpu.VMEM((chunk,), f32)]
        + [pltpu.VMEM((L,), f32)]
        + [pltpu.SemaphoreType.DMA] * 3
    )
    return pl.kernel(
        body,
        out_type=[
            jax.ShapeDtypeStruct((NC * 3 * n_pad,), f32),
            jax.ShapeDtypeStruct((NW * L,), f32),
        ],
        mesh=mesh,
        scratch_types=scratch,
    )


def kernel(pos, edge_index):
    n, _ = pos.shape
    e = edge_index.shape[1]
    align = NS * L
    n_pad = ((n + align - 1) // align) * align
    posp = jnp.pad(pos, ((0, n_pad - n), (0, 0)))
    px = posp[:, 0]
    py = posp[:, 1]
    pz = posp[:, 2]
    eflat = edge_index.reshape(2 * e)
    sck = _build_sc_kernel(n_pad, e)
    fout, eout = sck(px, py, pz, eflat)
    fout = fout.reshape(NC, 3, n_pad)
    forces = (fout[0] + fout[1])[:, :n].T
    energy = (0.5 * K) * jnp.sum(eout).reshape(1)
    return energy, forces
